# baseline (speedup 1.0000x reference)
import functools

import jax
import jax.numpy as jnp
from jax import lax
from jax.experimental import pallas as pl
from jax.experimental.pallas import tpu as pltpu
from jax.experimental.pallas import tpu_sc as plsc

EMB_DIM = 32


@functools.partial(jax.jit, static_argnums=(2, 3))
def _gather_sc(x_flat, table, B, CI):
    NW = 32
    b_per_w = B // NW
    n_ci = b_per_w // CI
    mesh = plsc.VectorSubcoreMesh(core_axis_name="c", subcore_axis_name="s")

    @functools.partial(
        pl.kernel,
        mesh=mesh,
        out_type=jax.ShapeDtypeStruct((B, EMB_DIM), jnp.float32),
        scratch_types=[
            pltpu.VMEM_SHARED((16, CI), jnp.int32),
            pltpu.SMEM((CI,), jnp.int32),
            pltpu.SemaphoreType.DMA,
        ],
        compiler_params=pltpu.CompilerParams(use_tc_tiling_on_sc=False),
    )
    def k(idx_hbm, table_hbm, out_hbm, idx_sh, idx_s, sem):
        sid = lax.axis_index("s")
        wid = sid * 2 + lax.axis_index("c")
        base = wid * b_per_w

        def outer(ci, carry):
            cbase = base + ci * CI
            pltpu.sync_copy(idx_hbm.at[pl.ds(cbase, CI)], idx_sh.at[sid])
            pltpu.sync_copy(idx_sh.at[sid], idx_s)

            def body(j, c):
                t = idx_s[j]
                pltpu.async_copy(
                    table_hbm.at[pl.ds(t, 1)],
                    out_hbm.at[pl.ds(cbase + j, 1)],
                    sem,
                )
                return c

            lax.fori_loop(0, CI, body, 0)
            pltpu.make_async_copy(
                table_hbm.at[pl.ds(0, CI)], out_hbm.at[pl.ds(cbase, CI)], sem
            ).wait()
            return carry

        lax.fori_loop(0, n_ci, outer, 0)

    return k(x_flat, table)


def kernel(x, table):
    B = x.shape[0] * x.shape[1]
    out = _gather_sc(x.reshape(B), table, B, 1600)
    return out.reshape(x.shape[0], x.shape[1], EMB_DIM)


# speedup vs baseline: 2.6884x; 2.6884x over previous
import functools

import jax
import jax.numpy as jnp
from jax import lax
from jax.experimental import pallas as pl
from jax.experimental.pallas import tpu as pltpu
from jax.experimental.pallas import tpu_sc as plsc

EMB_DIM = 32


@functools.partial(jax.jit, static_argnums=(2, 3, 4))
def _gather_sc(x_flat, table, B, C, dma_batch):
    NW = 32
    b_per_w = B // NW
    n_chunks = b_per_w // (C + dma_batch)
    n_stream = C * n_chunks
    n_dma = dma_batch * n_chunks
    mesh = plsc.VectorSubcoreMesh(core_axis_name="c", subcore_axis_name="s")

    @functools.partial(
        pl.kernel,
        mesh=mesh,
        out_type=jax.ShapeDtypeStruct((B, EMB_DIM), jnp.float32),
        scratch_types=[
            pltpu.VMEM((n_stream,), jnp.int32),
            pltpu.VMEM((C, EMB_DIM), jnp.float32),
            pltpu.VMEM((C, EMB_DIM), jnp.float32),
            pltpu.VMEM_SHARED((16, n_dma), jnp.int32),
            pltpu.SMEM((dma_batch,), jnp.int32),
            pltpu.SemaphoreType.DMA,
            pltpu.SemaphoreType.DMA,
            pltpu.SemaphoreType.DMA,
            pltpu.SemaphoreType.DMA,
            pltpu.SemaphoreType.DMA,
        ],
        compiler_params=pltpu.CompilerParams(use_tc_tiling_on_sc=False),
    )
    def k(idx_hbm, table_hbm, out_hbm, idx_v, rows0, rows1, idx_sh, idx_s,
          sg0, sg1, so0, so1, sd):
        sid = lax.axis_index("s")
        wid = sid * 2 + lax.axis_index("c")
        base = wid * b_per_w
        dma_base = base + n_stream

        pltpu.sync_copy(idx_hbm.at[pl.ds(dma_base, n_dma)], idx_sh.at[sid])
        pltpu.sync_copy(idx_hbm.at[pl.ds(base, n_stream)], idx_v)

        rows = (rows0, rows1)
        sg = (sg0, sg1)
        so = (so0, so1)

        def gather(i, b):
            pltpu.async_copy(table_hbm.at[idx_v.at[pl.ds(i * C, C)]], rows[b], sg[b])

        def wait_gather(b):
            pltpu.make_async_copy(
                table_hbm.at[idx_v.at[pl.ds(0, C)]], rows[b], sg[b]
            ).wait()

        def store(i, b):
            pltpu.async_copy(rows[b], out_hbm.at[pl.ds(base + i * C, C)], so[b])

        def wait_store(b):
            pltpu.make_async_copy(rows[b], out_hbm.at[pl.ds(base, C)], so[b]).wait()

        def dma_rows(i):
            lo = i * dma_batch

            def body(j, c):
                t = idx_s[j]
                pltpu.async_copy(
                    table_hbm.at[pl.ds(t, 1)],
                    out_hbm.at[pl.ds(dma_base + lo + j, 1)],
                    sd,
                )
                return c

            lax.fori_loop(0, dma_batch, body, 0)

        gather(0, 0)
        gather(1, 1)

        def body(p, carry):
            for b in range(2):
                i = p * 2 + b
                pltpu.sync_copy(
                    idx_sh.at[sid, pl.ds(i * dma_batch, dma_batch)], idx_s
                )
                dma_rows(i)
                wait_gather(b)
                store(i, b)
                wait_store(b)

                @pl.when(i + 2 < n_chunks)
                def _():
                    gather(i + 2, b)

            return carry

        lax.fori_loop(0, n_chunks // 2, body, 0)

        pltpu.make_async_copy(
            table_hbm.at[pl.ds(0, n_dma)], out_hbm.at[pl.ds(dma_base, n_dma)], sd
        ).wait()

    return k(x_flat, table)


def kernel(x, table):
    B = x.shape[0] * x.shape[1]
    out = _gather_sc(x.reshape(B), table, B, 648, 152)
    return out.reshape(x.shape[0], x.shape[1], EMB_DIM)


# speedup vs baseline: 4.1353x; 1.5382x over previous
import functools

import jax
import jax.numpy as jnp
from jax import lax
from jax.experimental import pallas as pl
from jax.experimental.pallas import tpu as pltpu
from jax.experimental.pallas import tpu_sc as plsc

EMB_DIM = 32


@functools.partial(jax.jit, static_argnums=(2, 3))
def _gather_sc(x_flat, table, B, C):
    NW = 32
    b_per_w = B // NW
    n_chunks = b_per_w // C
    n_pairs = n_chunks // 2
    mesh = plsc.VectorSubcoreMesh(core_axis_name="c", subcore_axis_name="s")

    @functools.partial(
        pl.kernel,
        mesh=mesh,
        out_type=jax.ShapeDtypeStruct((B, EMB_DIM), jnp.float32),
        scratch_types=[
            pltpu.VMEM((b_per_w,), jnp.int32),
            pltpu.VMEM((C, EMB_DIM), jnp.float32),
            pltpu.VMEM((C, EMB_DIM), jnp.float32),
            pltpu.SemaphoreType.DMA,
            pltpu.SemaphoreType.DMA,
            pltpu.SemaphoreType.DMA,
            pltpu.SemaphoreType.DMA,
            pltpu.SemaphoreType.DMA,
        ],
        compiler_params=pltpu.CompilerParams(use_tc_tiling_on_sc=False),
    )
    def k(idx_hbm, table_hbm, out_hbm, idx_v, rows0, rows1, sg0, sg1, so0, so1, si):
        wid = lax.axis_index("s") * 2 + lax.axis_index("c")
        base = wid * b_per_w

        rows = (rows0, rows1)
        sg = (sg0, sg1)
        so = (so0, so1)

        def idx_load(i):
            pltpu.async_copy(
                idx_hbm.at[pl.ds(base + i * C, C)], idx_v.at[pl.ds(i * C, C)], si
            )

        def wait_idx():
            pltpu.make_async_copy(
                idx_hbm.at[pl.ds(base, C)], idx_v.at[pl.ds(0, C)], si
            ).wait()

        def gather(i, b):
            pltpu.async_copy(table_hbm.at[idx_v.at[pl.ds(i * C, C)]], rows[b], sg[b])

        def wait_gather(b):
            pltpu.make_async_copy(
                table_hbm.at[idx_v.at[pl.ds(0, C)]], rows[b], sg[b]
            ).wait()

        def store(i, b):
            pltpu.async_copy(rows[b], out_hbm.at[pl.ds(base + i * C, C)], so[b])

        def wait_store(b):
            pltpu.make_async_copy(rows[b], out_hbm.at[pl.ds(base, C)], so[b]).wait()

        for i in range(min(4, n_chunks)):
            idx_load(i)
        for b in range(2):
            wait_idx()
            gather(b, b)

        def body(p, carry):
            for b in range(2):
                i = p * 2 + b
                wait_gather(b)
                store(i, b)
                wait_store(b)

                @pl.when(i + 4 < n_chunks)
                def _():
                    idx_load(i + 4)

                @pl.when(i + 2 < n_chunks)
                def _():
                    wait_idx()
                    gather(i + 2, b)

            return carry

        lax.fori_loop(0, n_pairs, body, 0)

    return k(x_flat, table)


def kernel(x, table):
    B = x.shape[0] * x.shape[1]
    out = _gather_sc(x.reshape(B), table, B, 1600)
    return out.reshape(x.shape[0], x.shape[1], EMB_DIM)
